# baseline (speedup 1.0000x reference)
import jax
import jax.numpy as jnp
from jax.experimental import pallas as pl
from jax.experimental.pallas import tpu as pltpu


def _lrelu(x):
    return jnp.where(x >= 0, x, 0.01 * x)


def _mlp2_body(x_ref, w1_ref, b1_ref, w2_ref, b2_ref, o_ref):
    h = jnp.dot(x_ref[...], w1_ref[...], preferred_element_type=jnp.float32)
    h = _lrelu(h + b1_ref[...])
    y = jnp.dot(h.astype(jnp.bfloat16), w2_ref[...],
                preferred_element_type=jnp.float32)
    o_ref[...] = y + b2_ref[...]


def _mlp2_dec_body(x_ref, s_ref, w1_ref, w1s_ref, b1_ref, w2_ref, b2_ref,
                   o_ref):
    h = jnp.dot(x_ref[...], w1_ref[...], preferred_element_type=jnp.float32)
    h = _lrelu(h + s_ref[...] * w1s_ref[...] + b1_ref[...])
    y = jnp.dot(h.astype(jnp.bfloat16), w2_ref[...],
                preferred_element_type=jnp.float32)
    o_ref[...] = y + b2_ref[...]


def _fused_mlp(x, w1, b1, w2, b2, tb):
    bsz, k = x.shape
    h = w1.shape[1]
    n = w2.shape[1]
    return pl.pallas_call(
        _mlp2_body,
        grid=(bsz // tb,),
        in_specs=[
            pl.BlockSpec((tb, k), lambda i: (i, 0)),
            pl.BlockSpec((k, h), lambda i: (0, 0)),
            pl.BlockSpec((1, h), lambda i: (0, 0)),
            pl.BlockSpec((h, n), lambda i: (0, 0)),
            pl.BlockSpec((1, n), lambda i: (0, 0)),
        ],
        out_specs=pl.BlockSpec((tb, n), lambda i: (i, 0)),
        out_shape=jax.ShapeDtypeStruct((bsz, n), jnp.float32),
        compiler_params=pltpu.CompilerParams(
            dimension_semantics=("parallel",)),
    )(x, w1, b1, w2, b2)


def _fused_mlp_dec(x, s, w1, w1s, b1, w2, b2, tb):
    bsz, k = x.shape
    h = w1.shape[1]
    n = w2.shape[1]
    return pl.pallas_call(
        _mlp2_dec_body,
        grid=(bsz // tb,),
        in_specs=[
            pl.BlockSpec((tb, k), lambda i: (i, 0)),
            pl.BlockSpec((tb, 1), lambda i: (i, 0)),
            pl.BlockSpec((k, h), lambda i: (0, 0)),
            pl.BlockSpec((1, h), lambda i: (0, 0)),
            pl.BlockSpec((1, h), lambda i: (0, 0)),
            pl.BlockSpec((h, n), lambda i: (0, 0)),
            pl.BlockSpec((1, n), lambda i: (0, 0)),
        ],
        out_specs=pl.BlockSpec((tb, n), lambda i: (i, 0)),
        out_shape=jax.ShapeDtypeStruct((bsz, n), jnp.float32),
        compiler_params=pltpu.CompilerParams(
            dimension_semantics=("parallel",)),
    )(x, s, w1, w1s, b1, w2, b2)


def kernel(data, s, mu_W1, mu_b1, mu_W2, mu_b2, lv_W1, lv_b1, lv_W2, lv_b2,
           dec_W1, dec_b1, dec_W2, dec_b2):
    bf = jnp.bfloat16
    tb = 512
    x16 = data.astype(bf)
    mu_h = _fused_mlp(x16, mu_W1.astype(bf), mu_b1.reshape(1, -1),
                      mu_W2.astype(bf), mu_b2.reshape(1, -1), tb)
    logvar_h = _fused_mlp(x16, lv_W1.astype(bf), lv_b1.reshape(1, -1),
                          lv_W2.astype(bf), lv_b2.reshape(1, -1), tb)
    data_reconst = _fused_mlp_dec(
        mu_h.astype(bf), s, dec_W1[:-1].astype(bf), dec_W1[-1:],
        dec_b1.reshape(1, -1), dec_W2.astype(bf), dec_b2.reshape(1, -1), tb)
    return (data_reconst, mu_h, logvar_h)
